# baseline (speedup 1.0000x reference)
import functools

import jax
import jax.numpy as jnp
import numpy as np
from jax import lax
from jax.experimental import pallas as pl
from jax.experimental.pallas import tpu as pltpu
from jax.experimental.pallas import tpu_sc as plsc

NN = 100000
EE = 1600000
GG = 1024
_DEG = np.asarray([10, 20, 50, 120, 280, 580, 1100, 1900, 3000, 4300, 5600,
                   6700, 7400, 7700, 7600, 7100, 6300, 5300, 4200, 3200, 2300,
                   1600, 1000, 600, 350, 190, 100, 50, 25, 12, 6, 3, 2],
                  dtype=np.float64)
AVG_LOG_C = float((np.log(np.arange(len(_DEG)) + 1.0) * _DEG).sum() / _DEG.sum())

NC_ = 2
NS_ = 16
LL = 16
NWK = NC_ * NS_
NPT = NN // NWK
NPT_PAD = 3136
SPAD = NPT_PAD + 16
E_PAD = EE + 1024
CE = 8000
CR = 1024
NB = 125
NBLK = NPT // NB
BIG = 3.0e38

_SC_CP = pltpu.CompilerParams(needs_layout_passes=False,
                              use_tc_tiling_on_sc=False)


def _wid():
    return lax.axis_index("s") * NC_ + lax.axis_index("c")


def _s1_body(dst_h, src_h, tab1_h, cnt_o, starts_o, csr_o, stats1_o,
             deg_v, starts_v, cur_v, cntf_v, dstc_v, srcc_v, posb_v, valb_v,
             csrc_v, rows1_v, stage1_v, sem):
    wid = _wid()
    base = wid * NPT
    zero16i = jnp.zeros((LL,), jnp.int32)
    zero16f = jnp.zeros((LL,), jnp.float32)
    big16 = jnp.full((LL,), BIG, jnp.float32)

    def a0(i, _):
        deg_v[pl.ds(i * LL, LL)] = zero16i
        return 0
    lax.fori_loop(0, NPT_PAD // LL, a0, 0)

    def a1_chunk(g, _):
        pltpu.sync_copy(dst_h.at[pl.ds(g * CE, CE)], dstc_v)

        def a1_grp(i, _2):
            d16 = dstc_v[pl.ds(i * LL, LL)]
            rel = d16 - base
            inb = (rel >= 0) & (rel < NPT)
            relc = jnp.where(inb, rel, NPT_PAD - 1)
            cnts, lastm = plsc.scan_count(relc)
            plsc.addupdate_scatter(deg_v, [relc], cnts, mask=lastm)
            return 0
        lax.fori_loop(0, CE // LL, a1_grp, 0, unroll=2)
        return 0
    lax.fori_loop(0, EE // CE, a1_chunk, 0)

    def a2(i, carry):
        d16 = deg_v[pl.ds(i * LL, LL)]
        cs = plsc.cumsum(d16)
        ex = (cs - d16) + carry
        starts_v[pl.ds(i * LL, LL)] = ex
        cur_v[pl.ds(i * LL, LL)] = ex
        cntf_v[pl.ds(i * LL, LL)] = d16.astype(jnp.float32)
        return carry + cs[LL - 1]
    lax.fori_loop(0, NPT_PAD // LL, a2, jnp.int32(0))
    starts_v[pl.ds(NPT_PAD, LL)] = jnp.full((LL,), 0x7FFFFFF, jnp.int32)
    pltpu.sync_copy(starts_v, starts_o.at[wid])
    pltpu.sync_copy(cntf_v, cnt_o.at[wid])

    dump = wid * E_PAD + (E_PAD - 1)

    def a3_chunk(g, _):
        pltpu.sync_copy(dst_h.at[pl.ds(g * CE, CE)], dstc_v)
        pltpu.sync_copy(src_h.at[pl.ds(g * CE, CE)], srcc_v)

        def a3_grp(i, _2):
            d16 = dstc_v[pl.ds(i * LL, LL)]
            s16 = srcc_v[pl.ds(i * LL, LL)]
            rel = d16 - base
            inb = (rel >= 0) & (rel < NPT)
            relc = jnp.where(inb, rel, NPT_PAD - 1)
            cnts, lastm = plsc.scan_count(relc)
            curs = plsc.load_gather(cur_v, [relc])
            pos = curs + cnts - 1
            plsc.store_scatter(cur_v, [relc], curs + cnts, mask=lastm)
            posb_v[pl.ds(i * LL, LL)] = jnp.where(inb, pos + wid * E_PAD, dump)
            valb_v[pl.ds(i * LL, LL)] = s16
            return 0
        lax.fori_loop(0, CE // LL, a3_grp, 0, unroll=2)
        pltpu.async_copy(valb_v, csr_o.at[posb_v], sem).wait()
        return 0
    lax.fori_loop(0, EE // CE, a3_chunk, 0)

    def blk(nb, _):
        nbb = nb * NB

        def sinit(jl, _2):
            stage1_v[jl, pl.ds(0, LL)] = zero16f
            stage1_v[jl, pl.ds(LL, LL)] = big16
            stage1_v[jl, pl.ds(2 * LL, LL)] = -big16
            return 0
        lax.fori_loop(0, NB, sinit, 0)

        s0 = starts_v[pl.ds(nbb, LL)][0]
        s1 = starts_v[pl.ds(nbb + NB, LL)][0]
        c0 = jnp.bitwise_and(s0, jnp.int32(-8))
        nch = (s1 - c0 + CR - 1) // CR

        def chunk(c, _2):
            cstart = c0 + c * CR
            off = pl.multiple_of(wid * E_PAD + cstart, 8)
            pltpu.sync_copy(csr_o.at[pl.ds(off, CR)], csrc_v)

            def clampg(t, _3):
                g16 = csrc_v[pl.ds(t * LL, LL)]
                csrc_v[pl.ds(t * LL, LL)] = jnp.clip(g16, 0, NN - 1)
                return 0
            lax.fori_loop(0, CR // LL, clampg, 0)
            pltpu.async_copy(tab1_h.at[csrc_v], rows1_v, sem).wait()

            def node(jl, _3):
                st_ = starts_v[pl.ds(nbb + jl, LL)][0]
                en_ = starts_v[pl.ds(nbb + jl + 1, LL)][0]
                lo = jnp.maximum(st_ - cstart, 0)
                hi = jnp.minimum(en_ - cstart, CR)
                hi = jnp.maximum(hi, lo)

                def edge(e, acc):
                    sm, mn, mx = acc
                    r = rows1_v[e, pl.ds(0, LL)]
                    return (sm + r, jnp.minimum(mn, r), jnp.maximum(mx, r))
                init = (stage1_v[jl, pl.ds(0, LL)],
                        stage1_v[jl, pl.ds(LL, LL)],
                        stage1_v[jl, pl.ds(2 * LL, LL)])
                sm, mn, mx = lax.fori_loop(lo, hi, edge, init)
                stage1_v[jl, pl.ds(0, LL)] = sm
                stage1_v[jl, pl.ds(LL, LL)] = mn
                stage1_v[jl, pl.ds(2 * LL, LL)] = mx
                return 0
            lax.fori_loop(0, NB, node, 0)
            return 0
        lax.fori_loop(0, nch, chunk, 0)
        pltpu.sync_copy(stage1_v, stats1_o.at[wid, pl.ds(nbb, NB)])
        return 0
    lax.fori_loop(0, NBLK, blk, 0)


def _make_s1():
    mesh = plsc.VectorSubcoreMesh(core_axis_name="c", subcore_axis_name="s")
    return functools.partial(
        pl.kernel, mesh=mesh, compiler_params=_SC_CP, name="pna_sc_csr_l1",
        out_type=[
            jax.ShapeDtypeStruct((NWK, NPT_PAD), jnp.float32),
            jax.ShapeDtypeStruct((NWK, SPAD), jnp.int32),
            jax.ShapeDtypeStruct((NWK * E_PAD,), jnp.int32),
            jax.ShapeDtypeStruct((NWK, NPT_PAD, 48), jnp.float32),
        ],
        scratch_types=[
            pltpu.VMEM((NPT_PAD,), jnp.int32),
            pltpu.VMEM((SPAD,), jnp.int32),
            pltpu.VMEM((NPT_PAD,), jnp.int32),
            pltpu.VMEM((NPT_PAD,), jnp.float32),
            pltpu.VMEM((CE,), jnp.int32),
            pltpu.VMEM((CE,), jnp.int32),
            pltpu.VMEM((CE,), jnp.int32),
            pltpu.VMEM((CE,), jnp.int32),
            pltpu.VMEM((CR,), jnp.int32),
            pltpu.VMEM((CR, LL), jnp.float32),
            pltpu.VMEM((NB, 48), jnp.float32),
            pltpu.SemaphoreType.DMA,
        ],
    )(_s1_body)


def _s2_body(starts_h, csr_h, tab2_h, stats2_o,
             starts_v, csrc_v, rows2_v, stage2_v, sem):
    wid = _wid()
    zero16f = jnp.zeros((LL,), jnp.float32)
    big16 = jnp.full((LL,), BIG, jnp.float32)
    pltpu.sync_copy(starts_h.at[wid], starts_v)

    def blk(nb, _):
        nbb = nb * NB

        def sinit(jl, _2):
            def wrt(k, _3):
                stage2_v[jl, pl.ds(k * LL, LL)] = zero16f
                stage2_v[jl, pl.ds(128 + k * LL, LL)] = big16
                stage2_v[jl, pl.ds(192 + k * LL, LL)] = -big16
                return 0
            lax.fori_loop(0, 8, wrt, 0)
            return 0
        lax.fori_loop(0, NB, sinit, 0)

        s0 = starts_v[pl.ds(nbb, LL)][0]
        s1 = starts_v[pl.ds(nbb + NB, LL)][0]
        c0 = jnp.bitwise_and(s0, jnp.int32(-8))
        nch = (s1 - c0 + CR - 1) // CR

        def chunk(c, _2):
            cstart = c0 + c * CR
            off = pl.multiple_of(wid * E_PAD + cstart, 8)
            pltpu.sync_copy(csr_h.at[pl.ds(off, CR)], csrc_v)

            def clampg(t, _3):
                g16 = csrc_v[pl.ds(t * LL, LL)]
                csrc_v[pl.ds(t * LL, LL)] = jnp.clip(g16, 0, NN - 1)
                return 0
            lax.fori_loop(0, CR // LL, clampg, 0)
            pltpu.async_copy(tab2_h.at[csrc_v], rows2_v, sem).wait()

            def node(jl, _3):
                st_ = starts_v[pl.ds(nbb + jl, LL)][0]
                en_ = starts_v[pl.ds(nbb + jl + 1, LL)][0]
                lo = jnp.maximum(st_ - cstart, 0)
                hi = jnp.minimum(en_ - cstart, CR)
                hi = jnp.maximum(hi, lo)

                def edge(e, acc):
                    (s_0, s_1, s_2, s_3, q0, q1, q2, q3,
                     m0, m1, m2, m3, x0, x1, x2, x3) = acc
                    r0 = rows2_v[e, pl.ds(0, LL)]
                    r1 = rows2_v[e, pl.ds(LL, LL)]
                    r2 = rows2_v[e, pl.ds(2 * LL, LL)]
                    r3 = rows2_v[e, pl.ds(3 * LL, LL)]
                    return (s_0 + r0, s_1 + r1, s_2 + r2, s_3 + r3,
                            q0 + r0 * r0, q1 + r1 * r1,
                            q2 + r2 * r2, q3 + r3 * r3,
                            jnp.minimum(m0, r0), jnp.minimum(m1, r1),
                            jnp.minimum(m2, r2), jnp.minimum(m3, r3),
                            jnp.maximum(x0, r0), jnp.maximum(x1, r1),
                            jnp.maximum(x2, r2), jnp.maximum(x3, r3))

                init = tuple(stage2_v[jl, pl.ds(k * LL, LL)] for k in range(16))
                res = lax.fori_loop(lo, hi, edge, init)

                def wb(k, _4):
                    return 0
                for k in range(16):
                    stage2_v[jl, pl.ds(k * LL, LL)] = res[k]
                return 0
            lax.fori_loop(0, NB, node, 0)
            return 0
        lax.fori_loop(0, nch, chunk, 0)
        pltpu.sync_copy(stage2_v, stats2_o.at[wid, pl.ds(nbb, NB)])
        return 0
    lax.fori_loop(0, NBLK, blk, 0)


def _make_s2():
    mesh = plsc.VectorSubcoreMesh(core_axis_name="c", subcore_axis_name="s")
    return functools.partial(
        pl.kernel, mesh=mesh, compiler_params=_SC_CP, name="pna_sc_l2",
        out_type=jax.ShapeDtypeStruct((NWK, NPT_PAD, 256), jnp.float32),
        scratch_types=[
            pltpu.VMEM((SPAD,), jnp.int32),
            pltpu.VMEM((CR,), jnp.int32),
            pltpu.VMEM((CR, 64), jnp.float32),
            pltpu.VMEM((NB, 256), jnp.float32),
            pltpu.SemaphoreType.DMA,
        ],
    )(_s2_body)


def _t0_body(w_ref, x_ref, out_ref):
    c1 = w_ref[1:2, 0:1]
    v = x_ref[...] * c1
    out_ref[...] = jnp.concatenate(
        [v, v * v, jnp.zeros((v.shape[0], 14), jnp.float32)], axis=1)


def _t1_body(x_ref, cnt_ref, st_ref, wpre_ref, bpre_ref, w13_ref, bp1_ref,
             wl1_ref, bl1_ref, b2p_ref, h_ref, v2_ref):
    x = x_ref[...]
    u = x * wpre_ref[0:1, 0:1] + bpre_ref[0:1, 0:1]
    S = st_ref[:, 0:1]
    SQ = st_ref[:, 1:2]
    MN = st_ref[:, 16:17]
    MX = st_ref[:, 32:33]
    c = cnt_ref[...]
    has = c > 0.0
    cc = jnp.maximum(c, 1.0)
    mean = jnp.where(has, u + S / cc, 0.0)
    var = SQ / cc - (S / cc) ** 2
    std = jnp.sqrt(jax.nn.relu(var) + 1e-5)
    mn = jnp.where(has, u + MN, 0.0)
    mx = jnp.where(has, u + MX, 0.0)
    lcc = jnp.log(cc + 1.0)
    ampf = lcc * (1.0 / AVG_LOG_C)
    attf = AVG_LOG_C / lcc
    z3 = jnp.zeros((x.shape[0], 3), jnp.float32)
    feats = jnp.concatenate(
        [x, mean, mn, mx, std,
         mean * ampf, mn * ampf, mx * ampf, std * ampf,
         mean * attf, mn * attf, mx * attf, std * attf, z3], axis=1)
    o1 = jnp.dot(feats, w13_ref[...], preferred_element_type=jnp.float32) + bp1_ref[...]
    h = jax.nn.relu(
        jnp.dot(o1, wl1_ref[...], preferred_element_type=jnp.float32) + bl1_ref[...])
    h_ref[...] = h
    v2_ref[...] = jnp.dot(h, b2p_ref[...], preferred_element_type=jnp.float32)


def _t2_body(h_ref, cnt_ref, st_ref, a2p_ref, bpre2_ref, wbig_ref, bp2_ref,
             wl2_ref, bl2_ref, out_ref):
    h = h_ref[...]
    u = jnp.dot(h, a2p_ref[...], preferred_element_type=jnp.float32) + bpre2_ref[...]
    S = st_ref[:, 0:64]
    SQ = st_ref[:, 64:128]
    MN = st_ref[:, 128:192]
    MX = st_ref[:, 192:256]
    c = cnt_ref[...]
    has = c > 0.0
    cc = jnp.maximum(c, 1.0)
    mean = jnp.where(has, u + S / cc, 0.0)
    var = SQ / cc - (S / cc) ** 2
    std = jnp.sqrt(jax.nn.relu(var) + 1e-5)
    mn = jnp.where(has, u + MN, 0.0)
    mx = jnp.where(has, u + MX, 0.0)
    lcc = jnp.log(cc + 1.0)
    ampf = lcc * (1.0 / AVG_LOG_C)
    attf = AVG_LOG_C / lcc
    feats = jnp.concatenate(
        [h, mean, mn, mx, std,
         mean * ampf, mn * ampf, mx * ampf, std * ampf,
         mean * attf, mn * attf, mx * attf, std * attf], axis=1)
    o = jnp.dot(feats, wbig_ref[...], preferred_element_type=jnp.float32) + bp2_ref[...]
    h2 = jax.nn.relu(
        jnp.dot(o, wl2_ref[...], preferred_element_type=jnp.float32) + bl2_ref[...])
    colid = lax.broadcasted_iota(jnp.int32, h2.shape, 1)
    out_ref[...] = jnp.where(colid == 50, 1.0, h2)


def _t3_body(bat_ref, h2_ref, out_ref):
    @pl.when(pl.program_id(0) == 0)
    def _():
        out_ref[...] = jnp.zeros_like(out_ref)
    bat = bat_ref[...]
    gid = lax.broadcasted_iota(jnp.int32, (bat.shape[0], GG), 1)
    oh = (bat == gid).astype(jnp.float32)
    out_ref[...] += lax.dot_general(
        oh, h2_ref[...], (((0,), (0,)), ((), ())),
        preferred_element_type=jnp.float32)


def _t4_body(gs_ref, wo_ref, bo_ref, out_ref):
    gs = gs_ref[...]
    cc = jnp.maximum(gs[:, 50:51], 1.0)
    pooled = gs / cc
    out_ref[...] = jax.nn.sigmoid(
        jnp.dot(pooled, wo_ref[...], preferred_element_type=jnp.float32)
        + bo_ref[0:1, 0:1])


def kernel(x, edge_index, batch, W_pre1, b_pre1, W_post1, b_post1, W_lin1,
           b_lin1, W_pre2, b_pre2, W_post2, b_post2, W_lin2, b_lin2, W_out,
           b_out):
    f32 = jnp.float32
    src = edge_index[0]
    dst = edge_index[1]

    w13 = jnp.zeros((16, 64), f32).at[0:13, 0:50].set(W_post1)
    bp1 = jnp.zeros((1, 64), f32).at[0, 0:50].set(b_post1)
    wl1 = jnp.zeros((64, 64), f32).at[0:50, 0:50].set(W_lin1)
    bl1 = jnp.zeros((1, 64), f32).at[0, 0:50].set(b_lin1)
    a2p = jnp.zeros((64, 64), f32).at[0:50, 0:50].set(W_pre2[0:50])
    b2p = jnp.zeros((64, 64), f32).at[0:50, 0:50].set(W_pre2[50:100])
    bpre2 = jnp.zeros((1, 64), f32).at[0, 0:50].set(b_pre2)
    wbig = jnp.zeros((832, 64), f32)
    for i in range(13):
        wbig = wbig.at[i * 64:i * 64 + 50, 0:50].set(W_post2[i * 50:(i + 1) * 50])
    bp2 = jnp.zeros((1, 64), f32).at[0, 0:50].set(b_post2)
    wl2 = jnp.zeros((64, 64), f32).at[0:50, 0:50].set(W_lin2)
    bl2 = jnp.zeros((1, 64), f32).at[0, 0:50].set(b_lin2)
    wo = jnp.zeros((64, 1), f32).at[0:50].set(W_out)
    bo = b_out.reshape(1, 1)
    wpre1 = W_pre1
    bpre1 = b_pre1.reshape(1, 1)

    BT0 = 10000
    tab1 = pl.pallas_call(
        _t0_body,
        grid=(NN // BT0,),
        in_specs=[pl.BlockSpec((2, 1), lambda i: (0, 0)),
                  pl.BlockSpec((BT0, 1), lambda i: (i, 0))],
        out_specs=pl.BlockSpec((BT0, 16), lambda i: (i, 0)),
        out_shape=jax.ShapeDtypeStruct((NN, 16), f32),
    )(wpre1, x)

    cnt_p, starts_p, csr, stats1_p = _make_s1()(dst, src, tab1)
    cnt = cnt_p[:, :NPT].reshape(NN, 1)
    stats1 = stats1_p[:, :NPT, :].reshape(NN, 48)

    BT1 = 4000
    h, tab2 = pl.pallas_call(
        _t1_body,
        grid=(NN // BT1,),
        in_specs=[pl.BlockSpec((BT1, 1), lambda i: (i, 0)),
                  pl.BlockSpec((BT1, 1), lambda i: (i, 0)),
                  pl.BlockSpec((BT1, 48), lambda i: (i, 0)),
                  pl.BlockSpec((2, 1), lambda i: (0, 0)),
                  pl.BlockSpec((1, 1), lambda i: (0, 0)),
                  pl.BlockSpec((16, 64), lambda i: (0, 0)),
                  pl.BlockSpec((1, 64), lambda i: (0, 0)),
                  pl.BlockSpec((64, 64), lambda i: (0, 0)),
                  pl.BlockSpec((1, 64), lambda i: (0, 0)),
                  pl.BlockSpec((64, 64), lambda i: (0, 0))],
        out_specs=[pl.BlockSpec((BT1, 64), lambda i: (i, 0)),
                   pl.BlockSpec((BT1, 64), lambda i: (i, 0))],
        out_shape=[jax.ShapeDtypeStruct((NN, 64), f32),
                   jax.ShapeDtypeStruct((NN, 64), f32)],
    )(x, cnt, stats1, wpre1, bpre1, w13, bp1, wl1, bl1, b2p)

    stats2_p = _make_s2()(starts_p, csr, tab2)
    stats2 = stats2_p[:, :NPT, :].reshape(NN, 256)

    BT2 = 4000
    h2e = pl.pallas_call(
        _t2_body,
        grid=(NN // BT2,),
        in_specs=[pl.BlockSpec((BT2, 64), lambda i: (i, 0)),
                  pl.BlockSpec((BT2, 1), lambda i: (i, 0)),
                  pl.BlockSpec((BT2, 256), lambda i: (i, 0)),
                  pl.BlockSpec((64, 64), lambda i: (0, 0)),
                  pl.BlockSpec((1, 64), lambda i: (0, 0)),
                  pl.BlockSpec((832, 64), lambda i: (0, 0)),
                  pl.BlockSpec((1, 64), lambda i: (0, 0)),
                  pl.BlockSpec((64, 64), lambda i: (0, 0)),
                  pl.BlockSpec((1, 64), lambda i: (0, 0))],
        out_specs=pl.BlockSpec((BT2, 64), lambda i: (i, 0)),
        out_shape=jax.ShapeDtypeStruct((NN, 64), f32),
    )(h, cnt, stats2, a2p, bpre2, wbig, bp2, wl2, bl2)

    BT3 = 2000
    bat2 = batch.reshape(NN, 1)
    gsums = pl.pallas_call(
        _t3_body,
        grid=(NN // BT3,),
        in_specs=[pl.BlockSpec((BT3, 1), lambda i: (i, 0)),
                  pl.BlockSpec((BT3, 64), lambda i: (i, 0))],
        out_specs=pl.BlockSpec((GG, 64), lambda i: (0, 0)),
        out_shape=jax.ShapeDtypeStruct((GG, 64), f32),
    )(bat2, h2e)

    out = pl.pallas_call(
        _t4_body,
        in_specs=[pl.BlockSpec((GG, 64), lambda: (0, 0)),
                  pl.BlockSpec((64, 1), lambda: (0, 0)),
                  pl.BlockSpec((1, 1), lambda: (0, 0))],
        out_specs=pl.BlockSpec((GG, 1), lambda: (0, 0)),
        out_shape=jax.ShapeDtypeStruct((GG, 1), f32),
    )(gsums, wo, bo)
    return out.reshape(-1)
